# baseline (speedup 1.0000x reference)
import functools

import jax
import jax.numpy as jnp
from jax import lax
from jax.experimental import pallas as pl
from jax.experimental.pallas import tpu as pltpu
from jax.experimental.pallas import tpu_sc as plsc

K_CODES = 1024
LATENT_DIM = 256
N_BATCH = 16
N_PIX = 576
N_TOTAL = N_BATCH * N_PIX
N_WORKERS = 32
D_PER_W = LATENT_DIM // N_WORKERS


def _vq_body(z_ref, w2_ref, zn_ref, wn_ref, iota_ref, idx_ref, prob_ref):
    zt = z_ref[0]
    W2 = w2_ref[...]
    k = W2.shape[0]
    mm2 = jax.lax.dot_general(
        zt, W2, (((0,), (1,)), ((), ())),
        preferred_element_type=jnp.float32,
    )
    zn_col = zn_ref[0]
    wn_row = wn_ref[...]
    dist = (zn_col + wn_row) - mm2
    rowmin = jnp.min(dist, axis=1, keepdims=True)
    e = jnp.exp(rowmin - dist)
    prob_ref[0] = e * (1.0 / jnp.sum(e, axis=1, keepdims=True))
    iota_row = iota_ref[...]
    masked = jnp.where(dist == rowmin, iota_row, float(k))
    idx_col = jnp.min(masked, axis=1, keepdims=True)
    idx_ref[0] = idx_col.astype(jnp.int32)


@functools.partial(
    pl.kernel,
    mesh=plsc.VectorSubcoreMesh(core_axis_name="c", subcore_axis_name="s"),
    out_type=jax.ShapeDtypeStruct((N_BATCH, LATENT_DIM, N_PIX), jnp.float32),
    compiler_params=pltpu.CompilerParams(
        use_tc_tiling_on_sc=False, needs_layout_passes=False),
    scratch_types=[
        pltpu.VMEM((D_PER_W * K_CODES,), jnp.float32),
        pltpu.VMEM((N_TOTAL,), jnp.int32),
        pltpu.VMEM((D_PER_W * N_TOTAL,), jnp.float32),
        pltpu.SemaphoreType.DMA,
    ],
)
def _sc_gather(wt_hbm, idx_hbm, out_hbm, wt_v, idx_v, rows_v, sem):
    wid = lax.axis_index("s") * 2 + lax.axis_index("c")
    dd_base = wid * D_PER_W
    pltpu.sync_copy(wt_hbm.at[pl.ds(dd_base * K_CODES, D_PER_W * K_CODES)], wt_v)
    pltpu.sync_copy(idx_hbm, idx_v)

    def body(i, carry):
        idxv = idx_v[pl.ds(i * 16, 16)]
        for dd in range(D_PER_W):
            rows_v[pl.ds(dd * N_TOTAL + i * 16, 16)] = plsc.load_gather(
                wt_v, [idxv + dd * K_CODES])
        return carry

    lax.fori_loop(0, N_TOTAL // 16, body, 0)
    copies = []
    for dd in range(D_PER_W):
        for b in range(N_BATCH):
            copies.append(pltpu.async_copy(
                rows_v.at[pl.ds(dd * N_TOTAL + b * N_PIX, N_PIX)],
                out_hbm.at[b, dd_base + dd], sem))
    for c in copies:
        c.wait()


@jax.jit
def kernel(z, W):
    b, d, h, w = z.shape
    n = h * w
    z_r = z.reshape(b, d, n)
    zn2 = jnp.sum(z_r ** 2, axis=1)[..., None]
    wn2 = jnp.sum(W ** 2, axis=1)
    w2 = W + W
    iota_row = jnp.arange(K_CODES, dtype=jnp.float32).reshape(1, K_CODES)
    idx, prob = pl.pallas_call(
        _vq_body,
        grid=(b,),
        in_specs=[
            pl.BlockSpec((1, d, n), lambda i: (i, 0, 0)),
            pl.BlockSpec((K_CODES, d), lambda i: (0, 0)),
            pl.BlockSpec((1, n, 1), lambda i: (i, 0, 0)),
            pl.BlockSpec((1, K_CODES), lambda i: (0, 0)),
            pl.BlockSpec((1, K_CODES), lambda i: (0, 0)),
        ],
        out_specs=[
            pl.BlockSpec((1, n, 1), lambda i: (i, 0, 0)),
            pl.BlockSpec((1, n, K_CODES), lambda i: (i, 0, 0)),
        ],
        out_shape=[
            jax.ShapeDtypeStruct((b, n, 1), jnp.int32),
            jax.ShapeDtypeStruct((b, n, K_CODES), jnp.float32),
        ],
    )(z_r, w2, zn2, wn2.reshape(1, K_CODES), iota_row)
    zq = _sc_gather(jnp.transpose(W).reshape(-1), idx.reshape(b * n))
    return (
        zq.reshape(b, d, h, w),
        idx.reshape(b * n),
        prob.reshape(b * n, K_CODES),
    )


# speedup vs baseline: 1.0694x; 1.0694x over previous
import functools

import jax
import jax.numpy as jnp
from jax import lax
from jax.experimental import pallas as pl
from jax.experimental.pallas import tpu as pltpu
from jax.experimental.pallas import tpu_sc as plsc

K_CODES = 1024
LATENT_DIM = 256
N_BATCH = 16
N_PIX = 576
N_TOTAL = N_BATCH * N_PIX
N_WORKERS = 32
D_PER_W = LATENT_DIM // N_WORKERS


def _vq_body(z_ref, w2_ref, zn_ref, wn_ref, iota_ref, idx_ref, prob_ref):
    zt = z_ref[0]
    W2 = w2_ref[...]
    k = W2.shape[0]
    mm2 = jax.lax.dot_general(
        zt, W2, (((0,), (1,)), ((), ())),
        preferred_element_type=jnp.float32,
    )
    zn_col = zn_ref[0]
    wn_row = wn_ref[...]
    dist = (zn_col + wn_row) - mm2
    rowmin = jnp.min(dist, axis=1, keepdims=True)
    e = jnp.exp(rowmin - dist)
    prob_ref[0] = e * (1.0 / jnp.sum(e, axis=1, keepdims=True))
    iota_row = iota_ref[...]
    masked = jnp.where(dist == rowmin, iota_row, float(k))
    idx_col = jnp.min(masked, axis=1, keepdims=True)
    idx_ref[0] = idx_col.astype(jnp.int32)


@functools.partial(
    pl.kernel,
    mesh=plsc.VectorSubcoreMesh(core_axis_name="c", subcore_axis_name="s"),
    out_type=jax.ShapeDtypeStruct((N_BATCH, LATENT_DIM, N_PIX), jnp.float32),
    compiler_params=pltpu.CompilerParams(
        use_tc_tiling_on_sc=False, needs_layout_passes=False),
    scratch_types=[
        pltpu.VMEM((K_CODES, D_PER_W), jnp.float32),
        pltpu.VMEM((N_TOTAL,), jnp.int32),
        pltpu.VMEM((D_PER_W, N_BATCH, N_PIX), jnp.float32),
        pltpu.SemaphoreType.DMA,
    ],
)
def _sc_gather(w_hbm, idx_hbm, out_hbm, wt_v, idx_v, rows_v, sem):
    wid = lax.axis_index("s") * 2 + lax.axis_index("c")
    dd_base = wid * D_PER_W
    pltpu.sync_copy(w_hbm.at[:, pl.ds(dd_base, D_PER_W)], wt_v)
    pltpu.sync_copy(idx_hbm, idx_v)

    for b in range(N_BATCH):
        @plsc.parallel_loop(0, N_PIX // 16, unroll=4)
        def _point_chunk(j, _b=b):
            idxv = idx_v[pl.ds(_b * N_PIX + j * 16, 16)]
            for dd in range(D_PER_W):
                rows_v[dd, _b, pl.ds(j * 16, 16)] = plsc.load_gather(
                    wt_v, [idxv, jnp.full((16,), dd, jnp.int32)])

    copies = [
        pltpu.async_copy(rows_v.at[dd], out_hbm.at[:, dd_base + dd], sem)
        for dd in range(D_PER_W)
    ]
    for c in copies:
        c.wait()


@jax.jit
def kernel(z, W):
    b, d, h, w = z.shape
    n = h * w
    z_r = z.reshape(b, d, n)
    zn2 = jnp.sum(z_r ** 2, axis=1)[..., None]
    wn2 = jnp.sum(W ** 2, axis=1)
    w2 = W + W
    iota_row = jnp.arange(K_CODES, dtype=jnp.float32).reshape(1, K_CODES)
    idx, prob = pl.pallas_call(
        _vq_body,
        grid=(b,),
        in_specs=[
            pl.BlockSpec((1, d, n), lambda i: (i, 0, 0)),
            pl.BlockSpec((K_CODES, d), lambda i: (0, 0)),
            pl.BlockSpec((1, n, 1), lambda i: (i, 0, 0)),
            pl.BlockSpec((1, K_CODES), lambda i: (0, 0)),
            pl.BlockSpec((1, K_CODES), lambda i: (0, 0)),
        ],
        out_specs=[
            pl.BlockSpec((1, n, 1), lambda i: (i, 0, 0)),
            pl.BlockSpec((1, n, K_CODES), lambda i: (i, 0, 0)),
        ],
        out_shape=[
            jax.ShapeDtypeStruct((b, n, 1), jnp.int32),
            jax.ShapeDtypeStruct((b, n, K_CODES), jnp.float32),
        ],
    )(z_r, w2, zn2, wn2.reshape(1, K_CODES), iota_row)
    zq = _sc_gather(W, idx.reshape(b * n))
    return (
        zq.reshape(b, d, h, w),
        idx.reshape(b * n),
        prob.reshape(b * n, K_CODES),
    )


# speedup vs baseline: 1.9711x; 1.8432x over previous
import jax
import jax.numpy as jnp
from jax.experimental import pallas as pl

K_CODES = 1024
LATENT_DIM = 256


def _vq_body(z_ref, w_ref, wn_ref, iota_ref, zq_ref, idx_ref, prob_ref):
    zt = z_ref[0]
    W = w_ref[...]
    k = W.shape[0]
    mm = jax.lax.dot_general(
        zt, W, (((0,), (1,)), ((), ())),
        preferred_element_type=jnp.float32,
    )
    zn_col = jnp.sum(zt * zt, axis=0)[:, None]
    wn_row = wn_ref[...]
    dist = (zn_col + wn_row) - 2.0 * mm
    rowmin = jnp.min(dist, axis=1, keepdims=True)
    e = jnp.exp(rowmin - dist)
    prob_ref[0] = e * (1.0 / jnp.sum(e, axis=1, keepdims=True))
    iota_row = iota_ref[...]
    masked = jnp.where(dist == rowmin, iota_row, float(k))
    idx_col = jnp.min(masked, axis=1, keepdims=True)
    idx_ref[0] = idx_col.astype(jnp.int32)
    onehot = jnp.where(iota_row == idx_col, 1.0, 0.0)
    zq_ref[0] = jax.lax.dot_general(
        W, onehot, (((0,), (1,)), ((), ())),
        preferred_element_type=jnp.float32,
    )


@jax.jit
def kernel(z, W):
    b, d, h, w = z.shape
    n = h * w
    z_r = z.reshape(b, d, n)
    wn2 = jnp.sum(W ** 2, axis=1)
    iota_row = jnp.arange(K_CODES, dtype=jnp.float32).reshape(1, K_CODES)
    zq, idx, prob = pl.pallas_call(
        _vq_body,
        grid=(b,),
        in_specs=[
            pl.BlockSpec((1, d, n), lambda i: (i, 0, 0)),
            pl.BlockSpec((K_CODES, d), lambda i: (0, 0)),
            pl.BlockSpec((1, K_CODES), lambda i: (0, 0)),
            pl.BlockSpec((1, K_CODES), lambda i: (0, 0)),
        ],
        out_specs=[
            pl.BlockSpec((1, d, n), lambda i: (i, 0, 0)),
            pl.BlockSpec((1, n, 1), lambda i: (i, 0, 0)),
            pl.BlockSpec((1, n, K_CODES), lambda i: (i, 0, 0)),
        ],
        out_shape=[
            jax.ShapeDtypeStruct((b, d, n), jnp.float32),
            jax.ShapeDtypeStruct((b, n, 1), jnp.int32),
            jax.ShapeDtypeStruct((b, n, K_CODES), jnp.float32),
        ],
    )(z_r, W, wn2.reshape(1, K_CODES), iota_row)
    return (
        zq.reshape(b, d, h, w),
        idx.reshape(b * n),
        prob.reshape(b * n, K_CODES),
    )
